# Initial kernel scaffold; baseline (speedup 1.0000x reference)
import jax
import jax.numpy as jnp
from jax.experimental import pallas as pl


def kernel(x, edge_index, W0, s0, W1, s1, W2, s2):
    raise NotImplementedError("write your pallas kernel here")


# speedup vs baseline: 4.4239x; 4.4239x over previous
import functools

import jax
import jax.numpy as jnp
from jax import lax
from jax.experimental import pallas as pl
from jax.experimental.pallas import tpu as pltpu
from jax.experimental.pallas import tpu_sc as plsc

N_NODES = 10000
N_EDGES = 320000
D = 128

NC = 2
NS = 16
NW = NC * NS
EDGES_PER_W = N_EDGES // NW
CHUNK = 80
N_CHUNKS = EDGES_PER_W // CHUNK
BAND = 624
TAIL = N_NODES - NS * BAND

R_BLK = 2000
N_BLK = N_NODES // R_BLK


def _project(h, s_scalar):
    h0 = h[:, 0:1]
    time = jax.nn.sigmoid(h0) * jnp.exp(s_scalar) + 1.1
    sq = jnp.maximum(jnp.sum(h * h, axis=1, keepdims=True) - h0 * h0, 1e-8)
    sfac = (time * time - 1.0) / sq
    out = h * jnp.sqrt(sfac)
    lane = lax.broadcasted_iota(jnp.int32, out.shape, 1)
    return jnp.where(lane == 0, time, out)


def _normalize(p):
    c0 = p[:, 0:1]
    inner = jnp.sum(p * p, axis=1, keepdims=True) - 2.0 * c0 * c0
    denom = jnp.sqrt(jnp.maximum(jnp.abs(inner), 1e-8))
    return p / denom


def _first_body(x_ref, w_ref, s_ref, o_ref):
    h = lax.dot_general(x_ref[...], w_ref[...], (((1,), (1,)), ((), ())),
                        precision=lax.Precision.HIGHEST,
                        preferred_element_type=jnp.float32)
    o_ref[...] = _project(h, s_ref[0])


def _mid_body(p0_ref, p1_ref, w_ref, s_ref, o_ref):
    hn = _normalize(p0_ref[...] + p1_ref[...])
    y = jnp.maximum(hn, 0.0)
    h = lax.dot_general(y, w_ref[...], (((1,), (1,)), ((), ())),
                        precision=lax.Precision.HIGHEST,
                        preferred_element_type=jnp.float32)
    o_ref[...] = _project(h, s_ref[0])


def _final_body(p0_ref, p1_ref, o_ref):
    o_ref[...] = _normalize(p0_ref[...] + p1_ref[...])


_row_spec = pl.BlockSpec((R_BLK, D), lambda i: (i, 0))
_w_spec = pl.BlockSpec((D, D), lambda i: (0, 0))
_s_spec = pl.BlockSpec(memory_space=pltpu.SMEM)
_out_shape = jax.ShapeDtypeStruct((N_NODES, D), jnp.float32)


def _linear_first(x, W, s):
    return pl.pallas_call(
        _first_body,
        grid=(N_BLK,),
        in_specs=[_row_spec, _w_spec, _s_spec],
        out_specs=_row_spec,
        out_shape=_out_shape,
    )(x, W, s.reshape(1))


def _linear_mid(p0, p1, W, s):
    return pl.pallas_call(
        _mid_body,
        grid=(N_BLK,),
        in_specs=[_row_spec, _row_spec, _w_spec, _s_spec],
        out_specs=_row_spec,
        out_shape=_out_shape,
    )(p0, p1, W, s.reshape(1))


def _norm_final(p0, p1):
    return pl.pallas_call(
        _final_body,
        grid=(N_BLK,),
        in_specs=[_row_spec, _row_spec],
        out_specs=_row_spec,
        out_shape=_out_shape,
    )(p0, p1)


def _sc_agg(h, src, dst, zeros):
    mesh = plsc.VectorSubcoreMesh(core_axis_name="c", subcore_axis_name="s")

    @functools.partial(
        pl.kernel,
        out_type=(jax.ShapeDtypeStruct((N_NODES, D), jnp.float32),
                  jax.ShapeDtypeStruct((N_NODES, D), jnp.float32)),
        mesh=mesh,
        scratch_types=[
            pltpu.VMEM((CHUNK,), jnp.int32),
            pltpu.VMEM((CHUNK,), jnp.int32),
            pltpu.VMEM((CHUNK, D), jnp.float32),
            pltpu.VMEM_SHARED((N_NODES, D), jnp.float32),
            pltpu.SemaphoreType.DMA,
        ],
    )
    def k(h_hbm, src_hbm, dst_hbm, z_hbm, out0, out1, sidx, didx, rows, acc, sem):
        c = lax.axis_index("c")
        s = lax.axis_index("s")
        wid = c * NS + s

        band = pl.ds(s * BAND, BAND)
        tail = pl.ds(NS * BAND, TAIL)
        pltpu.sync_copy(z_hbm.at[band], acc.at[band])

        @pl.when(s == 0)
        def _():
            pltpu.sync_copy(z_hbm.at[tail], acc.at[tail])

        plsc.subcore_barrier()

        def body(j, carry):
            base = pl.multiple_of(wid * EDGES_PER_W + j * CHUNK, CHUNK)
            pltpu.sync_copy(src_hbm.at[pl.ds(base, CHUNK)], sidx)
            pltpu.sync_copy(dst_hbm.at[pl.ds(base, CHUNK)], didx)
            pltpu.async_copy(h_hbm.at[sidx], rows, sem).wait()
            pltpu.sync_copy(rows, acc.at[didx], add=True)
            return carry

        lax.fori_loop(0, N_CHUNKS, body, 0)
        plsc.subcore_barrier()

        @pl.when(c == 0)
        def _():
            pltpu.sync_copy(acc.at[band], out0.at[band])

            @pl.when(s == 0)
            def _():
                pltpu.sync_copy(acc.at[tail], out0.at[tail])

        @pl.when(c == 1)
        def _():
            pltpu.sync_copy(acc.at[band], out1.at[band])

            @pl.when(s == 0)
            def _():
                pltpu.sync_copy(acc.at[tail], out1.at[tail])

    return k(h, src, dst, zeros)


def kernel(x, edge_index, W0, s0, W1, s1, W2, s2):
    src = edge_index[0]
    dst = edge_index[1]
    zeros = jnp.zeros((N_NODES, D), jnp.float32)

    h = _linear_first(x, W0, s0)
    p0, p1 = _sc_agg(h, src, dst, zeros)
    h = _linear_mid(p0, p1, W1, s1)
    p0, p1 = _sc_agg(h, src, dst, zeros)
    h = _linear_mid(p0, p1, W2, s2)
    p0, p1 = _sc_agg(h, src, dst, zeros)
    return _norm_final(p0, p1)


# speedup vs baseline: 8.6030x; 1.9447x over previous
import functools

import jax
import jax.numpy as jnp
from jax import lax
from jax.experimental import pallas as pl
from jax.experimental.pallas import tpu as pltpu
from jax.experimental.pallas import tpu_sc as plsc

N_NODES = 10000
N_EDGES = 320000
D = 128

NC = 2
NS = 16
NW = NC * NS
EDGES_PER_W = N_EDGES // NW
CHUNK = 50
N_CHUNKS = EDGES_PER_W // CHUNK
N_PAIRS = N_CHUNKS // 2
BAND = 624
TAIL = N_NODES - NS * BAND

R_BLK = 2000
N_BLK = N_NODES // R_BLK


def _project(h, s_scalar):
    h0 = h[:, 0:1]
    time = jax.nn.sigmoid(h0) * jnp.exp(s_scalar) + 1.1
    sq = jnp.maximum(jnp.sum(h * h, axis=1, keepdims=True) - h0 * h0, 1e-8)
    sfac = (time * time - 1.0) / sq
    out = h * jnp.sqrt(sfac)
    lane = lax.broadcasted_iota(jnp.int32, out.shape, 1)
    return jnp.where(lane == 0, time, out)


def _normalize(p):
    c0 = p[:, 0:1]
    inner = jnp.sum(p * p, axis=1, keepdims=True) - 2.0 * c0 * c0
    denom = jnp.sqrt(jnp.maximum(jnp.abs(inner), 1e-8))
    return p / denom


def _first_body(x_ref, w_ref, s_ref, o_ref):
    h = lax.dot_general(x_ref[...], w_ref[...], (((1,), (1,)), ((), ())),
                        precision=lax.Precision.HIGHEST,
                        preferred_element_type=jnp.float32)
    o_ref[...] = _project(h, s_ref[0])


def _mid_body(p0_ref, p1_ref, w_ref, s_ref, o_ref):
    hn = _normalize(p0_ref[...] + p1_ref[...])
    y = jnp.maximum(hn, 0.0)
    h = lax.dot_general(y, w_ref[...], (((1,), (1,)), ((), ())),
                        precision=lax.Precision.HIGHEST,
                        preferred_element_type=jnp.float32)
    o_ref[...] = _project(h, s_ref[0])


def _final_body(p0_ref, p1_ref, o_ref):
    o_ref[...] = _normalize(p0_ref[...] + p1_ref[...])


_row_spec = pl.BlockSpec((R_BLK, D), lambda i: (i, 0))
_w_spec = pl.BlockSpec((D, D), lambda i: (0, 0))
_s_spec = pl.BlockSpec(memory_space=pltpu.SMEM)
_out_shape = jax.ShapeDtypeStruct((N_NODES, D), jnp.float32)


def _linear_first(x, W, s):
    return pl.pallas_call(
        _first_body,
        grid=(N_BLK,),
        in_specs=[_row_spec, _w_spec, _s_spec],
        out_specs=_row_spec,
        out_shape=_out_shape,
    )(x, W, s.reshape(1))


def _linear_mid(p0, p1, W, s):
    return pl.pallas_call(
        _mid_body,
        grid=(N_BLK,),
        in_specs=[_row_spec, _row_spec, _w_spec, _s_spec],
        out_specs=_row_spec,
        out_shape=_out_shape,
    )(p0, p1, W, s.reshape(1))


def _norm_final(p0, p1):
    return pl.pallas_call(
        _final_body,
        grid=(N_BLK,),
        in_specs=[_row_spec, _row_spec],
        out_specs=_row_spec,
        out_shape=_out_shape,
    )(p0, p1)


def _sc_agg(h, src, dst, zeros):
    mesh = plsc.VectorSubcoreMesh(core_axis_name="c", subcore_axis_name="s")

    @functools.partial(
        pl.kernel,
        out_type=(jax.ShapeDtypeStruct((N_NODES, D), jnp.float32),
                  jax.ShapeDtypeStruct((N_NODES, D), jnp.float32)),
        mesh=mesh,
        scratch_types=[
            pltpu.VMEM((2, 2, 2, CHUNK), jnp.int32),
            pltpu.VMEM((2, 2, CHUNK, D), jnp.float32),
            pltpu.VMEM_SHARED((N_NODES, D), jnp.float32),
            pltpu.SemaphoreType.DMA,
            pltpu.SemaphoreType.DMA,
            pltpu.SemaphoreType.DMA,
            pltpu.SemaphoreType.DMA,
            pltpu.SemaphoreType.DMA,
            pltpu.SemaphoreType.DMA,
        ],
    )
    def k(h_hbm, src_hbm, dst_hbm, z_hbm, out0, out1,
          idxb, bufs, acc, sem_i0, sem_i1, sem_g0, sem_g1, sem_s0, sem_s1):
        c = lax.axis_index("c")
        s = lax.axis_index("s")
        wid = c * NS + s

        band = pl.ds(s * BAND, BAND)
        tail = pl.ds(NS * BAND, TAIL)
        pltpu.sync_copy(z_hbm.at[band], acc.at[band])

        @pl.when(s == 0)
        def _():
            pltpu.sync_copy(z_hbm.at[tail], acc.at[tail])

        plsc.subcore_barrier()

        sem_i = (sem_i0, sem_i1)
        sem_g = (sem_g0, sem_g1)
        sem_s = (sem_s0, sem_s1)

        def fire_idx(p, pb):
            for cc in range(2):
                pltpu.async_copy(src_hbm.at[wid, 2 * p + cc],
                                 idxb.at[pb, cc, 0], sem_i[pb])
                pltpu.async_copy(dst_hbm.at[wid, 2 * p + cc],
                                 idxb.at[pb, cc, 1], sem_i[pb])

        def drain_idx(p, pb):
            for cc in range(2):
                pltpu.make_async_copy(src_hbm.at[wid, 2 * p + cc],
                                      idxb.at[pb, cc, 0], sem_i[pb]).wait()
                pltpu.make_async_copy(dst_hbm.at[wid, 2 * p + cc],
                                      idxb.at[pb, cc, 1], sem_i[pb]).wait()

        def fire_gather(pb, cc):
            return pltpu.async_copy(
                h_hbm.at[idxb.at[pb, cc, 0]], bufs.at[pb, cc], sem_g[cc])

        def fire_scatter(pb, cc):
            pltpu.async_copy(bufs.at[pb, cc], acc.at[idxb.at[pb, cc, 1]],
                             sem_s[cc], add=True)

        def drain_scatter(pb, cc):
            pltpu.make_async_copy(bufs.at[pb, cc], acc.at[idxb.at[pb, cc, 1]],
                                  sem_s[cc]).wait()

        def step(p, pb, prefetch):
            drain_idx(p, pb)
            gd0 = fire_gather(pb, 0)
            gd1 = fire_gather(pb, 1)
            drain_scatter(1 - pb, 0)
            drain_scatter(1 - pb, 1)
            if prefetch:
                fire_idx(p + 1, 1 - pb)
            gd0.wait()
            fire_scatter(pb, 0)
            gd1.wait()
            fire_scatter(pb, 1)

        fire_idx(0, 0)
        drain_idx(0, 0)
        fire_idx(1, 1)
        gd0 = fire_gather(0, 0)
        gd1 = fire_gather(0, 1)
        gd0.wait()
        fire_scatter(0, 0)
        gd1.wait()
        fire_scatter(0, 1)

        def body(t, carry):
            step(2 * t + 1, 1, True)
            step(2 * t + 2, 0, True)
            return carry

        lax.fori_loop(0, (N_PAIRS - 2) // 2, body, 0)
        step(N_PAIRS - 1, 1, False)
        drain_scatter(1, 0)
        drain_scatter(1, 1)
        plsc.subcore_barrier()

        @pl.when(c == 0)
        def _():
            pltpu.sync_copy(acc.at[band], out0.at[band])

            @pl.when(s == 0)
            def _():
                pltpu.sync_copy(acc.at[tail], out0.at[tail])

        @pl.when(c == 1)
        def _():
            pltpu.sync_copy(acc.at[band], out1.at[band])

            @pl.when(s == 0)
            def _():
                pltpu.sync_copy(acc.at[tail], out1.at[tail])

    return k(h, src.reshape(NW, N_CHUNKS, CHUNK),
             dst.reshape(NW, N_CHUNKS, CHUNK), zeros)


def kernel(x, edge_index, W0, s0, W1, s1, W2, s2):
    src = edge_index[0]
    dst = edge_index[1]
    zeros = jnp.zeros((N_NODES, D), jnp.float32)

    h = _linear_first(x, W0, s0)
    p0, p1 = _sc_agg(h, src, dst, zeros)
    h = _linear_mid(p0, p1, W1, s1)
    p0, p1 = _sc_agg(h, src, dst, zeros)
    h = _linear_mid(p0, p1, W2, s2)
    p0, p1 = _sc_agg(h, src, dst, zeros)
    return _norm_final(p0, p1)


# speedup vs baseline: 8.6428x; 1.0046x over previous
import functools

import jax
import jax.numpy as jnp
from jax import lax
from jax.experimental import pallas as pl
from jax.experimental.pallas import tpu as pltpu
from jax.experimental.pallas import tpu_sc as plsc

N_NODES = 10000
N_EDGES = 320000
D = 128

NC = 2
NS = 16
NW = NC * NS
EDGES_PER_W = N_EDGES // NW
CHUNK = 100
N_CHUNKS = EDGES_PER_W // CHUNK
BAND = 624
TAIL = N_NODES - NS * BAND

R_BLK = 2000
N_BLK = N_NODES // R_BLK


def _project(h, s_scalar):
    h0 = h[:, 0:1]
    time = jax.nn.sigmoid(h0) * jnp.exp(s_scalar) + 1.1
    sq = jnp.maximum(jnp.sum(h * h, axis=1, keepdims=True) - h0 * h0, 1e-8)
    sfac = (time * time - 1.0) / sq
    out = h * jnp.sqrt(sfac)
    lane = lax.broadcasted_iota(jnp.int32, out.shape, 1)
    return jnp.where(lane == 0, time, out)


def _normalize(p):
    c0 = p[:, 0:1]
    inner = jnp.sum(p * p, axis=1, keepdims=True) - 2.0 * c0 * c0
    denom = jnp.sqrt(jnp.maximum(jnp.abs(inner), 1e-8))
    return p / denom


def _first_body(x_ref, w_ref, s_ref, o_ref):
    h = lax.dot_general(x_ref[...], w_ref[...], (((1,), (1,)), ((), ())),
                        precision=lax.Precision.HIGHEST,
                        preferred_element_type=jnp.float32)
    o_ref[...] = _project(h, s_ref[0])


def _mid_body(p0_ref, p1_ref, w_ref, s_ref, o_ref):
    hn = _normalize(p0_ref[...] + p1_ref[...])
    y = jnp.maximum(hn, 0.0)
    h = lax.dot_general(y, w_ref[...], (((1,), (1,)), ((), ())),
                        precision=lax.Precision.HIGHEST,
                        preferred_element_type=jnp.float32)
    o_ref[...] = _project(h, s_ref[0])


def _final_body(p0_ref, p1_ref, o_ref):
    o_ref[...] = _normalize(p0_ref[...] + p1_ref[...])


_row_spec = pl.BlockSpec((R_BLK, D), lambda i: (i, 0))
_w_spec = pl.BlockSpec((D, D), lambda i: (0, 0))
_s_spec = pl.BlockSpec(memory_space=pltpu.SMEM)
_out_shape = jax.ShapeDtypeStruct((N_NODES, D), jnp.float32)


def _linear_first(x, W, s):
    return pl.pallas_call(
        _first_body,
        grid=(N_BLK,),
        in_specs=[_row_spec, _w_spec, _s_spec],
        out_specs=_row_spec,
        out_shape=_out_shape,
    )(x, W, s.reshape(1))


def _linear_mid(p0, p1, W, s):
    return pl.pallas_call(
        _mid_body,
        grid=(N_BLK,),
        in_specs=[_row_spec, _row_spec, _w_spec, _s_spec],
        out_specs=_row_spec,
        out_shape=_out_shape,
    )(p0, p1, W, s.reshape(1))


def _norm_final(p0, p1):
    return pl.pallas_call(
        _final_body,
        grid=(N_BLK,),
        in_specs=[_row_spec, _row_spec],
        out_specs=_row_spec,
        out_shape=_out_shape,
    )(p0, p1)


def _sc_agg(h, ei, zeros):
    mesh = plsc.VectorSubcoreMesh(core_axis_name="c", subcore_axis_name="s")

    @functools.partial(
        pl.kernel,
        out_type=(jax.ShapeDtypeStruct((N_NODES, D), jnp.float32),
                  jax.ShapeDtypeStruct((N_NODES, D), jnp.float32)),
        mesh=mesh,
        scratch_types=[
            pltpu.VMEM((2, 2, CHUNK), jnp.int32),
            pltpu.VMEM((2, CHUNK, D), jnp.float32),
            pltpu.VMEM_SHARED((N_NODES, D), jnp.float32),
            pltpu.SemaphoreType.DMA,
            pltpu.SemaphoreType.DMA,
            pltpu.SemaphoreType.DMA,
            pltpu.SemaphoreType.DMA,
            pltpu.SemaphoreType.DMA,
            pltpu.SemaphoreType.DMA,
        ],
    )
    def k(h_hbm, ei_hbm, z_hbm, out0, out1,
          idxb, bufs, acc, sem_i0, sem_i1, sem_g0, sem_g1, sem_s0, sem_s1):
        c = lax.axis_index("c")
        s = lax.axis_index("s")
        wid = c * NS + s

        sem_i = (sem_i0, sem_i1)
        sem_g = (sem_g0, sem_g1)
        sem_s = (sem_s0, sem_s1)

        def fire_idx(g, b):
            return pltpu.async_copy(ei_hbm.at[wid, g], idxb.at[b], sem_i[b])

        def drain_idx(g, b):
            pltpu.make_async_copy(ei_hbm.at[wid, g], idxb.at[b],
                                  sem_i[b]).wait()

        def fire_gather(b):
            return pltpu.async_copy(
                h_hbm.at[idxb.at[b, 0]], bufs.at[b], sem_g[b])

        def fire_scatter(b):
            pltpu.async_copy(bufs.at[b], acc.at[idxb.at[b, 1]],
                             sem_s[b], add=True)

        def drain_scatter(b):
            pltpu.make_async_copy(bufs.at[b], acc.at[idxb.at[b, 1]],
                                  sem_s[b]).wait()

        fire_idx(0, 0)
        band = pl.ds(s * BAND, BAND)
        tail = pl.ds(NS * BAND, TAIL)
        pltpu.sync_copy(z_hbm.at[band], acc.at[band])

        @pl.when(s == 0)
        def _():
            pltpu.sync_copy(z_hbm.at[tail], acc.at[tail])

        plsc.subcore_barrier()

        drain_idx(0, 0)
        gd = fire_gather(0)
        fire_idx(1, 1)
        gd.wait()
        fire_scatter(0)

        def step(g, b, prefetch):
            drain_idx(g, b)
            gd = fire_gather(b)
            drain_scatter(1 - b)
            if prefetch:
                fire_idx(g + 1, 1 - b)
            gd.wait()
            fire_scatter(b)

        def body(t, carry):
            step(2 * t + 1, 1, True)
            step(2 * t + 2, 0, True)
            return carry

        lax.fori_loop(0, (N_CHUNKS - 2) // 2, body, 0)
        step(N_CHUNKS - 1, 1, False)
        drain_scatter(1)
        plsc.subcore_barrier()

        @pl.when(c == 0)
        def _():
            pltpu.sync_copy(acc.at[band], out0.at[band])

            @pl.when(s == 0)
            def _():
                pltpu.sync_copy(acc.at[tail], out0.at[tail])

        @pl.when(c == 1)
        def _():
            pltpu.sync_copy(acc.at[band], out1.at[band])

            @pl.when(s == 0)
            def _():
                pltpu.sync_copy(acc.at[tail], out1.at[tail])

    return k(h, ei, zeros)


def kernel(x, edge_index, W0, s0, W1, s1, W2, s2):
    ei = jnp.transpose(edge_index.reshape(2, NW, N_CHUNKS, CHUNK),
                       (1, 2, 0, 3))
    zeros = jnp.zeros((N_NODES, D), jnp.float32)

    h = _linear_first(x, W0, s0)
    p0, p1 = _sc_agg(h, ei, zeros)
    h = _linear_mid(p0, p1, W1, s1)
    p0, p1 = _sc_agg(h, ei, zeros)
    h = _linear_mid(p0, p1, W2, s2)
    p0, p1 = _sc_agg(h, ei, zeros)
    return _norm_final(p0, p1)


# speedup vs baseline: 9.9800x; 1.1547x over previous
import functools

import jax
import jax.numpy as jnp
from jax import lax
from jax.experimental import pallas as pl
from jax.experimental.pallas import tpu as pltpu
from jax.experimental.pallas import tpu_sc as plsc

N_NODES = 10000
N_EDGES = 320000
D = 128

NC = 2
NS = 16
NW = NC * NS
EDGES_PER_W = N_EDGES // NW
CHUNK = 80
N_CHUNKS = EDGES_PER_W // CHUNK
BAND = 624
TAIL = N_NODES - NS * BAND

R_BLK = 2000
N_BLK = N_NODES // R_BLK


def _project(h, s_scalar):
    h0 = h[:, 0:1]
    time = jax.nn.sigmoid(h0) * jnp.exp(s_scalar) + 1.1
    sq = jnp.maximum(jnp.sum(h * h, axis=1, keepdims=True) - h0 * h0, 1e-8)
    sfac = (time * time - 1.0) / sq
    out = h * jnp.sqrt(sfac)
    lane = lax.broadcasted_iota(jnp.int32, out.shape, 1)
    return jnp.where(lane == 0, time, out)


def _normalize(p):
    c0 = p[:, 0:1]
    inner = jnp.sum(p * p, axis=1, keepdims=True) - 2.0 * c0 * c0
    denom = jnp.sqrt(jnp.maximum(jnp.abs(inner), 1e-8))
    return p / denom


def _first_body(x_ref, w_ref, s_ref, o_ref):
    h = lax.dot_general(x_ref[...], w_ref[...], (((1,), (1,)), ((), ())),
                        precision=lax.Precision.HIGHEST,
                        preferred_element_type=jnp.float32)
    o_ref[...] = _project(h, s_ref[0])


def _mid_body(p0_ref, p1_ref, w_ref, s_ref, o_ref):
    hn = _normalize(p0_ref[...] + p1_ref[...])
    y = jnp.maximum(hn, 0.0)
    h = lax.dot_general(y, w_ref[...], (((1,), (1,)), ((), ())),
                        precision=lax.Precision.HIGHEST,
                        preferred_element_type=jnp.float32)
    o_ref[...] = _project(h, s_ref[0])


def _final_body(p0_ref, p1_ref, o_ref):
    o_ref[...] = _normalize(p0_ref[...] + p1_ref[...])


_row_spec = pl.BlockSpec((R_BLK, D), lambda i: (i, 0))
_w_spec = pl.BlockSpec((D, D), lambda i: (0, 0))
_s_spec = pl.BlockSpec(memory_space=pltpu.SMEM)
_out_shape = jax.ShapeDtypeStruct((N_NODES, D), jnp.float32)


def _linear_first(x, W, s):
    return pl.pallas_call(
        _first_body,
        grid=(N_BLK,),
        in_specs=[_row_spec, _w_spec, _s_spec],
        out_specs=_row_spec,
        out_shape=_out_shape,
    )(x, W, s.reshape(1))


def _linear_mid(p0, p1, W, s):
    return pl.pallas_call(
        _mid_body,
        grid=(N_BLK,),
        in_specs=[_row_spec, _row_spec, _w_spec, _s_spec],
        out_specs=_row_spec,
        out_shape=_out_shape,
    )(p0, p1, W, s.reshape(1))


def _norm_final(p0, p1):
    return pl.pallas_call(
        _final_body,
        grid=(N_BLK,),
        in_specs=[_row_spec, _row_spec],
        out_specs=_row_spec,
        out_shape=_out_shape,
    )(p0, p1)


def _sc_agg(h, ei, zeros):
    mesh = plsc.VectorSubcoreMesh(core_axis_name="c", subcore_axis_name="s")

    @functools.partial(
        pl.kernel,
        out_type=(jax.ShapeDtypeStruct((N_NODES, D), jnp.float32),
                  jax.ShapeDtypeStruct((N_NODES, D), jnp.float32)),
        mesh=mesh,
        scratch_types=[
            pltpu.VMEM((3, 2, CHUNK), jnp.int32),
            pltpu.VMEM((3, CHUNK, D), jnp.float32),
            pltpu.VMEM_SHARED((N_NODES, D), jnp.float32),
            pltpu.SemaphoreType.DMA,
            pltpu.SemaphoreType.DMA,
            pltpu.SemaphoreType.DMA,
            pltpu.SemaphoreType.DMA,
            pltpu.SemaphoreType.DMA,
            pltpu.SemaphoreType.DMA,
            pltpu.SemaphoreType.DMA,
            pltpu.SemaphoreType.DMA,
            pltpu.SemaphoreType.DMA,
        ],
    )
    def k(h_hbm, ei_hbm, z_hbm, out0, out1, idxb, bufs, acc, *sems):
        c = lax.axis_index("c")
        s = lax.axis_index("s")
        wid = c * NS + s

        sem_i = sems[0:3]
        sem_g = sems[3:6]
        sem_s = sems[6:9]

        def fire_idx(g, b):
            return pltpu.async_copy(ei_hbm.at[wid, g], idxb.at[b], sem_i[b])

        def drain_idx(g, b):
            pltpu.make_async_copy(ei_hbm.at[wid, g], idxb.at[b],
                                  sem_i[b]).wait()

        def fire_gather(b):
            return pltpu.async_copy(
                h_hbm.at[idxb.at[b, 0]], bufs.at[b], sem_g[b])

        def fire_scatter(b):
            pltpu.async_copy(bufs.at[b], acc.at[idxb.at[b, 1]],
                             sem_s[b], add=True)

        def drain_scatter(b):
            pltpu.make_async_copy(bufs.at[b], acc.at[idxb.at[b, 1]],
                                  sem_s[b]).wait()

        fire_idx(0, 0)
        band = pl.ds(s * BAND, BAND)
        tail = pl.ds(NS * BAND, TAIL)
        pltpu.sync_copy(z_hbm.at[band], acc.at[band])

        @pl.when(s == 0)
        def _():
            pltpu.sync_copy(z_hbm.at[tail], acc.at[tail])

        plsc.subcore_barrier()

        def drain_gather(b):
            pltpu.make_async_copy(h_hbm.at[idxb.at[b, 0]], bufs.at[b],
                                  sem_g[b]).wait()

        fire_idx(1, 1)
        drain_idx(0, 0)
        fire_gather(0)
        fire_idx(2, 2)
        drain_idx(1, 1)
        fire_gather(1)

        def step(g, b, pf_gather, pf_idx):
            drain_gather(b)
            fire_scatter(b)
            drain_scatter(b)
            if pf_gather:
                drain_idx(g + 2, (b + 2) % 3)
                fire_gather((b + 2) % 3)
            if pf_idx:
                fire_idx(g + 3, b)

        def body(t, carry):
            g = 3 * t
            step(g, 0, True, True)
            step(g + 1, 1, True, True)
            step(g + 2, 2, True, True)
            return carry

        lax.fori_loop(0, (N_CHUNKS - 5) // 3, body, 0)
        step(N_CHUNKS - 5, 0, True, True)
        step(N_CHUNKS - 4, 1, True, True)
        step(N_CHUNKS - 3, 2, True, False)
        step(N_CHUNKS - 2, 0, False, False)
        step(N_CHUNKS - 1, 1, False, False)
        plsc.subcore_barrier()

        @pl.when(c == 0)
        def _():
            pltpu.sync_copy(acc.at[band], out0.at[band])

            @pl.when(s == 0)
            def _():
                pltpu.sync_copy(acc.at[tail], out0.at[tail])

        @pl.when(c == 1)
        def _():
            pltpu.sync_copy(acc.at[band], out1.at[band])

            @pl.when(s == 0)
            def _():
                pltpu.sync_copy(acc.at[tail], out1.at[tail])

    return k(h, ei, zeros)


def kernel(x, edge_index, W0, s0, W1, s1, W2, s2):
    ei = jnp.transpose(edge_index.reshape(2, NW, N_CHUNKS, CHUNK),
                       (1, 2, 0, 3))
    zeros = jnp.zeros((N_NODES, D), jnp.float32)

    h = _linear_first(x, W0, s0)
    p0, p1 = _sc_agg(h, ei, zeros)
    h = _linear_mid(p0, p1, W1, s1)
    p0, p1 = _sc_agg(h, ei, zeros)
    h = _linear_mid(p0, p1, W2, s2)
    p0, p1 = _sc_agg(h, ei, zeros)
    return _norm_final(p0, p1)
